# baseline (speedup 1.0000x reference)
import jax
import jax.numpy as jnp
from jax import lax
from jax.experimental import pallas as pl
from jax.experimental.pallas import tpu as pltpu
from jax.experimental.pallas import tpu_sc as plsc

_BATCH = 4096
_SEQ = 200
_DIM = 64
_LANES = 16
_NC = 2
_NS = 16
_NW = _NC * _NS
_BPW = _BATCH // _NW
_NCH = _DIM // _LANES
_EPC = 2
_CROWS = _EPC * _SEQ
_NCHUNK = _BPW // _EPC
_NBUF = 2


def _encode_body(idx_hbm, table_hbm, out_hbm, idx_v, rows_v, out_v, sems):
    wid = lax.axis_index("s") * _NC + lax.axis_index("c")
    base = wid * _BPW
    pltpu.sync_copy(idx_hbm.at[pl.ds(base * _SEQ, _BPW * _SEQ)], idx_v)

    def start(c, b):
        pltpu.async_copy(
            table_hbm.at[idx_v.at[pl.ds(c * _CROWS, _CROWS)]],
            rows_v.at[b],
            sems.at[b],
        )

    def drain(c, b):
        pltpu.make_async_copy(
            table_hbm.at[idx_v.at[pl.ds(c * _CROWS, _CROWS)]],
            rows_v.at[b],
            sems.at[b],
        ).wait()

    def accum(e, b, h):
        for c in range(_NCH):
            out_v[e, pl.ds(_LANES * c, _LANES)] = jnp.zeros(
                (_LANES,), jnp.float32)

        @pl.loop(h * _SEQ, (h + 1) * _SEQ, step=8)
        def _row(s):
            for s2 in range(8):
                for c in range(_NCH):
                    plsc.addupdate(
                        out_v.at[e, pl.ds(_LANES * c, _LANES)],
                        rows_v[b, s + s2, pl.ds(_LANES * c, _LANES)],
                    )

        scale = jnp.float32(1.0 / _SEQ)
        for c in range(_NCH):
            out_v[e, pl.ds(_LANES * c, _LANES)] = (
                out_v[e, pl.ds(_LANES * c, _LANES)] * scale)

    for b in range(_NBUF):
        start(b, b)

    @pl.loop(0, _NCHUNK, step=_NBUF)
    def _chunk(ch):
        for b in range(_NBUF):
            cc = ch + b
            drain(cc, b)
            for h in range(_EPC):
                accum(_EPC * cc + h, b, h)

            @pl.when(cc + _NBUF < _NCHUNK)
            def _prefetch():
                start(cc + _NBUF, b)

    pltpu.sync_copy(out_v, out_hbm.at[pl.ds(base, _BPW)])


def kernel(token_ids, table):
    idx_flat = token_ids.astype(jnp.int32).reshape(_BATCH * _SEQ)
    mesh = plsc.VectorSubcoreMesh(core_axis_name="c", subcore_axis_name="s")
    k = pl.kernel(
        _encode_body,
        out_type=jax.ShapeDtypeStruct((_BATCH, _DIM), jnp.float32),
        mesh=mesh,
        compiler_params=pltpu.CompilerParams(use_tc_tiling_on_sc=False),
        scratch_types=[
            pltpu.VMEM((_BPW * _SEQ,), jnp.int32),
            pltpu.VMEM((_NBUF, _CROWS, _DIM), jnp.float32),
            pltpu.VMEM((_BPW, _DIM), jnp.float32),
            pltpu.SemaphoreType.DMA((_NBUF,)),
        ],
    )
    return k(idx_flat, table)


# speedup vs baseline: 1.0020x; 1.0020x over previous
import jax
import jax.numpy as jnp
from jax import lax
from jax.experimental import pallas as pl
from jax.experimental.pallas import tpu as pltpu
from jax.experimental.pallas import tpu_sc as plsc

_BATCH = 4096
_SEQ = 200
_DIM = 64
_LANES = 16
_NC = 2
_NS = 16
_NW = _NC * _NS
_BPW = _BATCH // _NW
_NCH = _DIM // _LANES
_EPC = 1
_CROWS = _EPC * _SEQ
_NCHUNK = _BPW // _EPC
_NBUF = 4


def _encode_body(idx_hbm, table_hbm, out_hbm, idx_v, rows_v, out_v, sems):
    wid = lax.axis_index("s") * _NC + lax.axis_index("c")
    base = wid * _BPW
    pltpu.sync_copy(idx_hbm.at[pl.ds(base * _SEQ, _BPW * _SEQ)], idx_v)

    def start(c, b):
        pltpu.async_copy(
            table_hbm.at[idx_v.at[pl.ds(c * _CROWS, _CROWS)]],
            rows_v.at[b],
            sems.at[b],
        )

    def drain(c, b):
        pltpu.make_async_copy(
            table_hbm.at[idx_v.at[pl.ds(c * _CROWS, _CROWS)]],
            rows_v.at[b],
            sems.at[b],
        ).wait()

    def accum(e, b, h):
        for c in range(_NCH):
            out_v[e, pl.ds(_LANES * c, _LANES)] = jnp.zeros(
                (_LANES,), jnp.float32)

        @pl.loop(h * _SEQ, (h + 1) * _SEQ, step=8)
        def _row(s):
            for s2 in range(8):
                for c in range(_NCH):
                    plsc.addupdate(
                        out_v.at[e, pl.ds(_LANES * c, _LANES)],
                        rows_v[b, s + s2, pl.ds(_LANES * c, _LANES)],
                    )

        scale = jnp.float32(1.0 / _SEQ)
        for c in range(_NCH):
            out_v[e, pl.ds(_LANES * c, _LANES)] = (
                out_v[e, pl.ds(_LANES * c, _LANES)] * scale)

    for b in range(_NBUF):
        start(b, b)

    @pl.loop(0, _NCHUNK, step=_NBUF)
    def _chunk(ch):
        for b in range(_NBUF):
            cc = ch + b
            drain(cc, b)
            for h in range(_EPC):
                accum(_EPC * cc + h, b, h)

            @pl.when(cc + _NBUF < _NCHUNK)
            def _prefetch():
                start(cc + _NBUF, b)

    pltpu.sync_copy(out_v, out_hbm.at[pl.ds(base, _BPW)])


def kernel(token_ids, table):
    idx_flat = token_ids.astype(jnp.int32).reshape(_BATCH * _SEQ)
    mesh = plsc.VectorSubcoreMesh(core_axis_name="c", subcore_axis_name="s")
    k = pl.kernel(
        _encode_body,
        out_type=jax.ShapeDtypeStruct((_BATCH, _DIM), jnp.float32),
        mesh=mesh,
        compiler_params=pltpu.CompilerParams(use_tc_tiling_on_sc=False),
        scratch_types=[
            pltpu.VMEM((_BPW * _SEQ,), jnp.int32),
            pltpu.VMEM((_NBUF, _CROWS, _DIM), jnp.float32),
            pltpu.VMEM((_BPW, _DIM), jnp.float32),
            pltpu.SemaphoreType.DMA((_NBUF,)),
        ],
    )
    return k(idx_flat, table)


# speedup vs baseline: 1.3071x; 1.3045x over previous
import jax
import jax.numpy as jnp
from jax import lax
from jax.experimental import pallas as pl
from jax.experimental.pallas import tpu as pltpu
from jax.experimental.pallas import tpu_sc as plsc

_BATCH = 4096
_SEQ = 200
_DIM = 64
_PDIM = 128
_LANES = 16
_NC = 2
_NS = 16
_NW = _NC * _NS
_BPW = _BATCH // _NW
_NCH = _DIM // _LANES
_NBUF = 2
_UNROLL = 4


def _encode_body(idx_hbm, table_hbm, out_hbm, idx_v, rows_v, out_v, sems):
    wid = lax.axis_index("s") * _NC + lax.axis_index("c")
    base = wid * _BPW
    pltpu.sync_copy(idx_hbm.at[pl.ds(base * _SEQ, _BPW * _SEQ)], idx_v)

    def start(e, b):
        pltpu.async_copy(
            table_hbm.at[idx_v.at[pl.ds(e * _SEQ, _SEQ)]],
            rows_v.at[b],
            sems.at[b],
        )

    def drain(e, b):
        pltpu.make_async_copy(
            table_hbm.at[idx_v.at[pl.ds(e * _SEQ, _SEQ)]],
            rows_v.at[b],
            sems.at[b],
        ).wait()

    def accum(e, b):
        def body(i, acc):
            s = i * _UNROLL
            for u in range(_UNROLL):
                acc = tuple(
                    acc[c] + rows_v[b, s + u, pl.ds(_LANES * c, _LANES)]
                    for c in range(_NCH))
            return acc

        zero = jnp.zeros((_LANES,), jnp.float32)
        acc = lax.fori_loop(0, _SEQ // _UNROLL, body, (zero,) * _NCH,
                            unroll=False)
        scale = jnp.float32(1.0 / _SEQ)
        for c in range(_NCH):
            out_v[e, pl.ds(_LANES * c, _LANES)] = acc[c] * scale

    for b in range(_NBUF):
        start(b, b)

    @pl.loop(0, _BPW, step=_NBUF)
    def _elem(e):
        for b in range(_NBUF):
            ee = e + b
            drain(ee, b)
            accum(ee, b)

            @pl.when(ee + _NBUF < _BPW)
            def _prefetch():
                start(ee + _NBUF, b)

    pltpu.sync_copy(out_v, out_hbm.at[pl.ds(base, _BPW)])


def kernel(token_ids, table):
    idx_flat = token_ids.astype(jnp.int32).reshape(_BATCH * _SEQ)
    table_pad = jnp.pad(table, ((0, 0), (0, _PDIM - _DIM)))
    mesh = plsc.VectorSubcoreMesh(core_axis_name="c", subcore_axis_name="s")
    k = pl.kernel(
        _encode_body,
        out_type=jax.ShapeDtypeStruct((_BATCH, _DIM), jnp.float32),
        mesh=mesh,
        compiler_params=pltpu.CompilerParams(use_tc_tiling_on_sc=True),
        scratch_types=[
            pltpu.VMEM((_BPW * _SEQ,), jnp.int32),
            pltpu.VMEM((_NBUF, _SEQ, _PDIM), jnp.float32),
            pltpu.VMEM((_BPW, _DIM), jnp.float32),
            pltpu.SemaphoreType.DMA((_NBUF,)),
        ],
    )
    return k(idx_flat, table_pad)
